# Initial kernel scaffold; baseline (speedup 1.0000x reference)
import jax
import jax.numpy as jnp
from jax.experimental import pallas as pl


def kernel(x, graph_edge_index, edge_index, W1, b1, W2, b2, Wc, bc):
    raise NotImplementedError("write your pallas kernel here")


# speedup vs baseline: 4.3887x; 4.3887x over previous
import functools

import jax
import jax.numpy as jnp
from jax import lax
from jax.experimental import pallas as pl
from jax.experimental.pallas import tpu as pltpu
from jax.experimental.pallas import tpu_sc as plsc

N = 10000
E = 320000
EQ = 320000
D = 128
H = 128
OUT = 2

NC = 2
NS = 16
NW = NC * NS
EPT = E // NW
BB = 128
NB = 80
EPTP = NB * BB
NA = 10240
RPT = NA // NS
VB = 16
NV = EPT // VB
TRASH = N

_mesh = plsc.VectorSubcoreMesh(
    core_axis_name="c", subcore_axis_name="s", num_cores=NC, num_subcores=NS
)
_sc_params = pltpu.CompilerParams(needs_layout_passes=False)


def _wid():
    return lax.axis_index("s") * NC + lax.axis_index("c")


@functools.partial(
    pl.kernel,
    out_type=(
        jax.ShapeDtypeStruct((NW * N,), jnp.float32),
        jax.ShapeDtypeStruct((NW * N,), jnp.float32),
    ),
    mesh=_mesh,
    compiler_params=_sc_params,
    scratch_types=[
        pltpu.VMEM((EPT,), jnp.int32),
        pltpu.VMEM((EPT,), jnp.int32),
        pltpu.VMEM((N,), jnp.float32),
        pltpu.VMEM((N,), jnp.float32),
    ],
)
def _k1_hist(src_f, dst_f, out_s, out_d, idx_s, idx_d, hist_s, hist_d):
    w = _wid()
    pltpu.sync_copy(src_f.at[pl.ds(w * EPT, EPT)], idx_s)
    pltpu.sync_copy(dst_f.at[pl.ds(w * EPT, EPT)], idx_d)

    z16 = jnp.zeros((VB,), jnp.float32)

    def zero_body(i, _):
        hist_s[pl.ds(i * VB, VB)] = z16
        hist_d[pl.ds(i * VB, VB)] = z16
        return 0

    lax.fori_loop(0, N // VB, zero_body, 0)

    ones16 = jnp.ones((VB,), jnp.float32)

    def body(i, _):
        s16 = idx_s[pl.ds(i * VB, VB)]
        d16 = idx_d[pl.ds(i * VB, VB)]
        plsc.addupdate_scatter(hist_s, [s16], ones16)
        plsc.addupdate_scatter(hist_d, [d16], ones16)
        return 0

    lax.fori_loop(0, NV, body, 0)

    pltpu.sync_copy(hist_s, out_s.at[pl.ds(w * N, N)])
    pltpu.sync_copy(hist_d, out_d.at[pl.ds(w * N, N)])


@functools.partial(
    pl.kernel,
    out_type=jax.ShapeDtypeStruct((NC, NA, D), jnp.float32),
    mesh=_mesh,
    compiler_params=_sc_params,
    scratch_types=[
        pltpu.VMEM((NB, BB), jnp.int32),
        pltpu.VMEM((NB, BB), jnp.int32),
        pltpu.VMEM((BB, D), jnp.float32),
        pltpu.VMEM_SHARED((NA, D), jnp.float32),
        pltpu.SemaphoreType.DMA,
    ],
)
def _k3_scatter(t, src3, dst3, zrows, acc_out, idx_s, idx_d, rows, acc_sh, sem):
    c = lax.axis_index("c")
    s = lax.axis_index("s")
    w = s * NC + c
    pltpu.sync_copy(src3.at[w], idx_s)
    pltpu.sync_copy(dst3.at[w], idx_d)

    pltpu.sync_copy(zrows, acc_sh.at[pl.ds(s * RPT, RPT)])
    plsc.subcore_barrier()

    def body(j, _):
        pltpu.async_copy(t.at[idx_s.at[j]], rows, sem).wait()
        pltpu.sync_copy(rows, acc_sh.at[idx_d.at[j]], add=True)
        return 0

    lax.fori_loop(0, NB, body, 0)
    plsc.subcore_barrier()

    pltpu.sync_copy(acc_sh.at[pl.ds(s * RPT, RPT)], acc_out.at[c, pl.ds(s * RPT, RPT)])


@functools.partial(
    pl.kernel,
    out_type=jax.ShapeDtypeStruct((EQ * OUT,), jnp.float32),
    mesh=_mesh,
    compiler_params=_sc_params,
    scratch_types=[
        pltpu.VMEM((N * 4,), jnp.float32),
        pltpu.VMEM((EPT,), jnp.int32),
        pltpu.VMEM((EPT,), jnp.int32),
        pltpu.VMEM((EPT * OUT,), jnp.float32),
    ],
)
def _k7_classify(ptab_f, qs_f, qd_f, out, ptab_v, qs_v, qd_v, out_v):
    w = _wid()
    pltpu.sync_copy(ptab_f, ptab_v)
    pltpu.sync_copy(qs_f.at[pl.ds(w * EPT, EPT)], qs_v)
    pltpu.sync_copy(qd_f.at[pl.ds(w * EPT, EPT)], qd_v)

    lanes = lax.iota(jnp.int32, VB)
    one = jnp.ones((VB,), jnp.float32)

    def body(i, _):
        s16 = qs_v[pl.ds(i * VB, VB)] * 4
        d16 = qd_v[pl.ds(i * VB, VB)] * 4
        a0 = plsc.load_gather(ptab_v, [s16])
        a1 = plsc.load_gather(ptab_v, [s16 + 1])
        b0 = plsc.load_gather(ptab_v, [d16 + 2])
        b1 = plsc.load_gather(ptab_v, [d16 + 3])
        p0 = one / (one + jnp.exp(-(a0 + b0)))
        p1 = one / (one + jnp.exp(-(a1 + b1)))
        base2 = (i * VB + lanes) * OUT
        plsc.store_scatter(out_v, [base2], p0)
        plsc.store_scatter(out_v, [base2 + 1], p1)
        return 0

    lax.fori_loop(0, NV, body, 0)
    pltpu.sync_copy(out_v, out.at[pl.ds(w * EPT * OUT, EPT * OUT)])


BN = 2000
GRID = N // BN


def _norm_from(parts):
    deg = jnp.sum(parts[0], axis=0)
    return lax.rsqrt(jnp.clip(deg, 1.0, None))


_HIST_SPEC = pl.BlockSpec((1, NW, BN), lambda j: (j, 0, 0))
_ACC_SPEC = pl.BlockSpec((NC, BN, H), lambda j: (0, j, 0))


def _k2_body(x_ref, hs_ref, w_ref, t_ref):
    ns = _norm_from(hs_ref[...])
    t_ref[...] = jnp.dot(
        x_ref[...] * ns[:, None], w_ref[...], preferred_element_type=jnp.float32
    )


def _k2_matmul(x, hs_part, W1):
    return pl.pallas_call(
        _k2_body,
        grid=(GRID,),
        in_specs=[
            pl.BlockSpec((BN, D), lambda j: (j, 0)),
            _HIST_SPEC,
            pl.BlockSpec((D, H), lambda j: (0, 0)),
        ],
        out_specs=pl.BlockSpec((BN, H), lambda j: (j, 0)),
        out_shape=jax.ShapeDtypeStruct((N, H), jnp.float32),
    )(x, hs_part, W1)


def _k4_body(acc_ref, hd_ref, hs_ref, b_ref, w_ref, t_ref):
    a = acc_ref[0] + acc_ref[1]
    nd = _norm_from(hd_ref[...])
    h = jnp.maximum(a * nd[:, None] + b_ref[...], 0.0)
    ns = _norm_from(hs_ref[...])
    t_ref[...] = jnp.dot(
        h * ns[:, None], w_ref[...], preferred_element_type=jnp.float32
    )


def _k4_mid(acc, hd_part, hs_part, b1, W2):
    return pl.pallas_call(
        _k4_body,
        grid=(GRID,),
        in_specs=[
            _ACC_SPEC,
            _HIST_SPEC,
            _HIST_SPEC,
            pl.BlockSpec((1, H), lambda j: (0, 0)),
            pl.BlockSpec((H, H), lambda j: (0, 0)),
        ],
        out_specs=pl.BlockSpec((BN, H), lambda j: (j, 0)),
        out_shape=jax.ShapeDtypeStruct((N, H), jnp.float32),
    )(acc, hd_part, hs_part, b1, W2)


def _k6_body(acc_ref, hd_ref, b_ref, wc1_ref, wc2_ref, bc_ref, h_ref, p_ref):
    a = acc_ref[0] + acc_ref[1]
    nd = _norm_from(hd_ref[...])
    h = jnp.maximum(a * nd[:, None] + b_ref[...], 0.0)
    h_ref[...] = h
    p1 = jnp.dot(h, wc1_ref[...], preferred_element_type=jnp.float32) + bc_ref[...]
    p2 = jnp.dot(h, wc2_ref[...], preferred_element_type=jnp.float32)
    p_ref[...] = jnp.concatenate([p1, p2], axis=1)


def _k6_final(acc, hd_part, b2, Wc1, Wc2, bc):
    return pl.pallas_call(
        _k6_body,
        grid=(GRID,),
        in_specs=[
            _ACC_SPEC,
            _HIST_SPEC,
            pl.BlockSpec((1, H), lambda j: (0, 0)),
            pl.BlockSpec((H, OUT), lambda j: (0, 0)),
            pl.BlockSpec((H, OUT), lambda j: (0, 0)),
            pl.BlockSpec((1, OUT), lambda j: (0, 0)),
        ],
        out_specs=[
            pl.BlockSpec((BN, H), lambda j: (j, 0)),
            pl.BlockSpec((BN, 2 * OUT), lambda j: (j, 0)),
        ],
        out_shape=[
            jax.ShapeDtypeStruct((N, H), jnp.float32),
            jax.ShapeDtypeStruct((N, 2 * OUT), jnp.float32),
        ],
    )(acc, hd_part, b2, Wc1, Wc2, bc)


def _pad_edges(idx, fill):
    chunks = idx.reshape(NW, EPT)
    pad = jnp.full((NW, EPTP - EPT), fill, jnp.int32)
    return jnp.concatenate([chunks, pad], axis=1).reshape(NW, NB, BB)


def kernel(x, graph_edge_index, edge_index, W1, b1, W2, b2, Wc, bc):
    src = graph_edge_index[0]
    dst = graph_edge_index[1]
    src3 = _pad_edges(src, 0)
    dst3 = _pad_edges(dst, TRASH)
    zrows = jnp.zeros((RPT, D), jnp.float32)

    hs_flat, hd_flat = _k1_hist(src, dst)
    hs_part = hs_flat.reshape(NW, GRID, BN).transpose(1, 0, 2)
    hd_part = hd_flat.reshape(NW, GRID, BN).transpose(1, 0, 2)

    t1 = _k2_matmul(x, hs_part, W1)
    acc1 = _k3_scatter(t1, src3, dst3, zrows)[:, :N]
    t2 = _k4_mid(acc1, hd_part, hs_part, b1.reshape(1, H), W2)
    acc2 = _k3_scatter(t2, src3, dst3, zrows)[:, :N]
    h2, ptab = _k6_final(
        acc2, hd_part, b2.reshape(1, H), Wc[:H], Wc[H:], bc.reshape(1, OUT)
    )
    probs = _k7_classify(
        ptab.reshape(N * 4), edge_index[0], edge_index[1]
    ).reshape(EQ, OUT)
    return (h2, probs)


# speedup vs baseline: 5.4438x; 1.2404x over previous
import functools

import jax
import jax.numpy as jnp
from jax import lax
from jax.experimental import pallas as pl
from jax.experimental.pallas import tpu as pltpu
from jax.experimental.pallas import tpu_sc as plsc

N = 10000
E = 320000
EQ = 320000
D = 128
H = 128
OUT = 2

NC = 2
NS = 16
NW = NC * NS
EPT = E // NW
BB = 128
NB = 80
EPTP = NB * BB
NBUF = 4
NA = 10240
RPT = NA // NS
VB = 16
NV = EPT // VB
TRASH = N

_mesh = plsc.VectorSubcoreMesh(
    core_axis_name="c", subcore_axis_name="s", num_cores=NC, num_subcores=NS
)
_sc_params = pltpu.CompilerParams(needs_layout_passes=False)


def _wid():
    return lax.axis_index("s") * NC + lax.axis_index("c")


@functools.partial(
    pl.kernel,
    out_type=(
        jax.ShapeDtypeStruct((NW * N,), jnp.float32),
        jax.ShapeDtypeStruct((NW * N,), jnp.float32),
    ),
    mesh=_mesh,
    compiler_params=_sc_params,
    scratch_types=[
        pltpu.VMEM((EPT,), jnp.int32),
        pltpu.VMEM((EPT,), jnp.int32),
        pltpu.VMEM((N,), jnp.float32),
        pltpu.VMEM((N,), jnp.float32),
    ],
)
def _k1_hist(src_f, dst_f, out_s, out_d, idx_s, idx_d, hist_s, hist_d):
    w = _wid()
    pltpu.sync_copy(src_f.at[pl.ds(w * EPT, EPT)], idx_s)
    pltpu.sync_copy(dst_f.at[pl.ds(w * EPT, EPT)], idx_d)

    z16 = jnp.zeros((VB,), jnp.float32)

    def zero_body(i, _):
        hist_s[pl.ds(i * VB, VB)] = z16
        hist_d[pl.ds(i * VB, VB)] = z16
        return 0

    lax.fori_loop(0, N // VB, zero_body, 0)

    ones16 = jnp.ones((VB,), jnp.float32)

    def body(i, _):
        s16 = idx_s[pl.ds(i * VB, VB)]
        d16 = idx_d[pl.ds(i * VB, VB)]
        plsc.addupdate_scatter(hist_s, [s16], ones16)
        plsc.addupdate_scatter(hist_d, [d16], ones16)
        return 0

    lax.fori_loop(0, NV, body, 0)

    pltpu.sync_copy(hist_s, out_s.at[pl.ds(w * N, N)])
    pltpu.sync_copy(hist_d, out_d.at[pl.ds(w * N, N)])


@functools.partial(
    pl.kernel,
    out_type=jax.ShapeDtypeStruct((NC, NA, D), jnp.float32),
    mesh=_mesh,
    compiler_params=_sc_params,
    scratch_types=[
        pltpu.VMEM((NB, BB), jnp.int32),
        pltpu.VMEM((NB, BB), jnp.int32),
        pltpu.VMEM((BB, D), jnp.float32),
        pltpu.VMEM_SHARED((NA, D), jnp.float32),
        pltpu.SemaphoreType.DMA,
    ],
)
def _k3_scatter(t, src3, dst3, zrows, acc_out, idx_s, idx_d, rows, acc_sh, sem):
    c = lax.axis_index("c")
    s = lax.axis_index("s")
    w = s * NC + c
    pltpu.sync_copy(src3.at[w], idx_s)
    pltpu.sync_copy(dst3.at[w], idx_d)

    pltpu.sync_copy(zrows, acc_sh.at[pl.ds(s * RPT, RPT)])
    plsc.subcore_barrier()

    def body(j, _):
        pltpu.async_copy(t.at[idx_s.at[j]], rows, sem).wait()
        pltpu.sync_copy(rows, acc_sh.at[idx_d.at[j]], add=True)
        return 0

    lax.fori_loop(0, NB, body, 0)
    plsc.subcore_barrier()

    pltpu.sync_copy(acc_sh.at[pl.ds(s * RPT, RPT)], acc_out.at[c, pl.ds(s * RPT, RPT)])


@functools.partial(
    pl.kernel,
    out_type=jax.ShapeDtypeStruct((EQ * OUT,), jnp.float32),
    mesh=_mesh,
    compiler_params=_sc_params,
    scratch_types=[
        pltpu.VMEM((N * 4,), jnp.float32),
        pltpu.VMEM((EPT,), jnp.int32),
        pltpu.VMEM((EPT,), jnp.int32),
        pltpu.VMEM((EPT * OUT,), jnp.float32),
    ],
)
def _k7_classify(ptab_f, qs_f, qd_f, out, ptab_v, qs_v, qd_v, out_v):
    w = _wid()
    pltpu.sync_copy(ptab_f, ptab_v)
    pltpu.sync_copy(qs_f.at[pl.ds(w * EPT, EPT)], qs_v)
    pltpu.sync_copy(qd_f.at[pl.ds(w * EPT, EPT)], qd_v)

    one = jnp.ones((VB,), jnp.float32)

    def body(i, _):
        s16 = qs_v[pl.ds(i * VB, VB)] * 4
        d16 = qd_v[pl.ds(i * VB, VB)] * 4
        a0 = plsc.load_gather(ptab_v, [s16])
        a1 = plsc.load_gather(ptab_v, [s16 + 1])
        b0 = plsc.load_gather(ptab_v, [d16 + 2])
        b1 = plsc.load_gather(ptab_v, [d16 + 3])
        out_v[pl.ds(i * VB, VB)] = one / (one + jnp.exp(-(a0 + b0)))
        out_v[pl.ds(EPT + i * VB, VB)] = one / (one + jnp.exp(-(a1 + b1)))
        return 0

    lax.fori_loop(0, NV, body, 0)
    pltpu.sync_copy(out_v.at[pl.ds(0, EPT)], out.at[pl.ds(w * EPT, EPT)])
    pltpu.sync_copy(out_v.at[pl.ds(EPT, EPT)], out.at[pl.ds(EQ + w * EPT, EPT)])


BN = 2000
GRID = N // BN


def _norm_from(parts):
    deg = jnp.sum(parts[0], axis=0)
    return lax.rsqrt(jnp.clip(deg, 1.0, None))


_HIST_SPEC = pl.BlockSpec((1, NW, BN), lambda j: (j, 0, 0))
_ACC_SPEC = pl.BlockSpec((NC, BN, H), lambda j: (0, j, 0))


def _k2_body(x_ref, hs_ref, w_ref, t_ref):
    ns = _norm_from(hs_ref[...])
    t_ref[...] = jnp.dot(
        x_ref[...] * ns[:, None], w_ref[...], preferred_element_type=jnp.float32
    )


def _k2_matmul(x, hs_part, W1):
    return pl.pallas_call(
        _k2_body,
        grid=(GRID,),
        in_specs=[
            pl.BlockSpec((BN, D), lambda j: (j, 0)),
            _HIST_SPEC,
            pl.BlockSpec((D, H), lambda j: (0, 0)),
        ],
        out_specs=pl.BlockSpec((BN, H), lambda j: (j, 0)),
        out_shape=jax.ShapeDtypeStruct((N, H), jnp.float32),
    )(x, hs_part, W1)


def _k4_body(acc_ref, hd_ref, hs_ref, b_ref, w_ref, t_ref):
    a = acc_ref[0] + acc_ref[1]
    nd = _norm_from(hd_ref[...])
    h = jnp.maximum(a * nd[:, None] + b_ref[...], 0.0)
    ns = _norm_from(hs_ref[...])
    t_ref[...] = jnp.dot(
        h * ns[:, None], w_ref[...], preferred_element_type=jnp.float32
    )


def _k4_mid(acc, hd_part, hs_part, b1, W2):
    return pl.pallas_call(
        _k4_body,
        grid=(GRID,),
        in_specs=[
            _ACC_SPEC,
            _HIST_SPEC,
            _HIST_SPEC,
            pl.BlockSpec((1, H), lambda j: (0, 0)),
            pl.BlockSpec((H, H), lambda j: (0, 0)),
        ],
        out_specs=pl.BlockSpec((BN, H), lambda j: (j, 0)),
        out_shape=jax.ShapeDtypeStruct((N, H), jnp.float32),
    )(acc, hd_part, hs_part, b1, W2)


def _k6_body(acc_ref, hd_ref, b_ref, wc1_ref, wc2_ref, bc_ref, h_ref, p_ref):
    a = acc_ref[0] + acc_ref[1]
    nd = _norm_from(hd_ref[...])
    h = jnp.maximum(a * nd[:, None] + b_ref[...], 0.0)
    h_ref[...] = h
    p1 = jnp.dot(h, wc1_ref[...], preferred_element_type=jnp.float32) + bc_ref[...]
    p2 = jnp.dot(h, wc2_ref[...], preferred_element_type=jnp.float32)
    p_ref[...] = jnp.concatenate([p1, p2], axis=1)


def _k6_final(acc, hd_part, b2, Wc1, Wc2, bc):
    return pl.pallas_call(
        _k6_body,
        grid=(GRID,),
        in_specs=[
            _ACC_SPEC,
            _HIST_SPEC,
            pl.BlockSpec((1, H), lambda j: (0, 0)),
            pl.BlockSpec((H, OUT), lambda j: (0, 0)),
            pl.BlockSpec((H, OUT), lambda j: (0, 0)),
            pl.BlockSpec((1, OUT), lambda j: (0, 0)),
        ],
        out_specs=[
            pl.BlockSpec((BN, H), lambda j: (j, 0)),
            pl.BlockSpec((BN, 2 * OUT), lambda j: (j, 0)),
        ],
        out_shape=[
            jax.ShapeDtypeStruct((N, H), jnp.float32),
            jax.ShapeDtypeStruct((N, 2 * OUT), jnp.float32),
        ],
    )(acc, hd_part, b2, Wc1, Wc2, bc)


def _pad_edges(idx, fill):
    chunks = idx.reshape(NW, EPT)
    pad = jnp.full((NW, EPTP - EPT), fill, jnp.int32)
    return jnp.concatenate([chunks, pad], axis=1).reshape(NW, NB, BB)


def kernel(x, graph_edge_index, edge_index, W1, b1, W2, b2, Wc, bc):
    src = graph_edge_index[0]
    dst = graph_edge_index[1]
    src3 = _pad_edges(src, 0)
    dst3 = _pad_edges(dst, TRASH)
    zrows = jnp.zeros((RPT, D), jnp.float32)

    hs_flat, hd_flat = _k1_hist(src, dst)
    hs_part = hs_flat.reshape(NW, GRID, BN).transpose(1, 0, 2)
    hd_part = hd_flat.reshape(NW, GRID, BN).transpose(1, 0, 2)

    t1 = _k2_matmul(x, hs_part, W1)
    acc1 = _k3_scatter(t1, src3, dst3, zrows)
    t2 = _k4_mid(acc1, hd_part, hs_part, b1.reshape(1, H), W2)
    acc2 = _k3_scatter(t2, src3, dst3, zrows)
    h2, ptab = _k6_final(
        acc2, hd_part, b2.reshape(1, H), Wc[:H], Wc[H:], bc.reshape(1, OUT)
    )
    pf = _k7_classify(ptab.reshape(N * 4), edge_index[0], edge_index[1])
    probs = jnp.stack([pf[:EQ], pf[EQ:]], axis=1)
    return (h2, probs)


# speedup vs baseline: 5.7757x; 1.0610x over previous
import functools

import jax
import jax.numpy as jnp
from jax import lax
from jax.experimental import pallas as pl
from jax.experimental.pallas import tpu as pltpu
from jax.experimental.pallas import tpu_sc as plsc

N = 10000
E = 320000
EQ = 320000
D = 128
H = 128
OUT = 2

NC = 2
NS = 16
NW = NC * NS
EPT = E // NW
BB = 128
NB = 80
CH = 8
EPTP = NB * BB
NA = 10112
RPT = NA // NS
VB = 16
NV = EPT // VB
TRASH = N

_mesh = plsc.VectorSubcoreMesh(
    core_axis_name="c", subcore_axis_name="s", num_cores=NC, num_subcores=NS
)
_sc_params = pltpu.CompilerParams(needs_layout_passes=False)


def _wid():
    return lax.axis_index("s") * NC + lax.axis_index("c")


@functools.partial(
    pl.kernel,
    out_type=(
        jax.ShapeDtypeStruct((NW * N,), jnp.float32),
        jax.ShapeDtypeStruct((NW * N,), jnp.float32),
    ),
    mesh=_mesh,
    compiler_params=_sc_params,
    scratch_types=[
        pltpu.VMEM((EPT,), jnp.int32),
        pltpu.VMEM((EPT,), jnp.int32),
        pltpu.VMEM((N,), jnp.float32),
        pltpu.VMEM((N,), jnp.float32),
    ],
)
def _k1_hist(src_f, dst_f, out_s, out_d, idx_s, idx_d, hist_s, hist_d):
    w = _wid()
    pltpu.sync_copy(src_f.at[pl.ds(w * EPT, EPT)], idx_s)
    pltpu.sync_copy(dst_f.at[pl.ds(w * EPT, EPT)], idx_d)

    z16 = jnp.zeros((VB,), jnp.float32)

    def zero_body(i, _):
        hist_s[pl.ds(i * VB, VB)] = z16
        hist_d[pl.ds(i * VB, VB)] = z16
        return 0

    lax.fori_loop(0, N // VB, zero_body, 0)

    ones16 = jnp.ones((VB,), jnp.float32)

    def body(i, _):
        s16 = idx_s[pl.ds(i * VB, VB)]
        d16 = idx_d[pl.ds(i * VB, VB)]
        plsc.addupdate_scatter(hist_s, [s16], ones16)
        plsc.addupdate_scatter(hist_d, [d16], ones16)
        return 0

    lax.fori_loop(0, NV, body, 0)

    pltpu.sync_copy(hist_s, out_s.at[pl.ds(w * N, N)])
    pltpu.sync_copy(hist_d, out_d.at[pl.ds(w * N, N)])


@functools.partial(
    pl.kernel,
    out_type=jax.ShapeDtypeStruct((NC, NA, D), jnp.float32),
    mesh=_mesh,
    compiler_params=_sc_params,
    scratch_types=[
        pltpu.VMEM((CH, BB), jnp.int32),
        pltpu.VMEM((CH, BB), jnp.int32),
        pltpu.VMEM((BB, D), jnp.float32),
        pltpu.VMEM((BB, D), jnp.float32),
        pltpu.VMEM_SHARED((NA, D), jnp.float32),
        pltpu.SemaphoreType.DMA,
    ],
)
def _k3_scatter(
    t, src3, dst3, zrows, acc_out,
    idx_s, idx_d, r0, r1, acc_sh, s0,
):
    c = lax.axis_index("c")
    s = lax.axis_index("s")
    w = s * NC + c

    pltpu.sync_copy(zrows, acc_sh.at[pl.ds(s * RPT, RPT)])
    plsc.subcore_barrier()

    def drain(buf):
        pltpu.make_async_copy(t.at[idx_s.at[0]], buf, s0).wait()

    def body(g, _):
        pltpu.sync_copy(src3.at[w, pl.ds(g * CH, CH)], idx_s)
        pltpu.sync_copy(dst3.at[w, pl.ds(g * CH, CH)], idx_d)
        for k in range(CH // 2):
            pltpu.sync_copy(t.at[idx_s.at[2 * k]], r0)
            if k > 0:
                drain(r1)
            pltpu.async_copy(r0, acc_sh.at[idx_d.at[2 * k]], s0, add=True)
            pltpu.sync_copy(t.at[idx_s.at[2 * k + 1]], r1)
            drain(r0)
            pltpu.async_copy(r1, acc_sh.at[idx_d.at[2 * k + 1]], s0, add=True)
        drain(r1)
        return 0

    lax.fori_loop(0, NB // CH, body, 0)
    plsc.subcore_barrier()

    pltpu.sync_copy(acc_sh.at[pl.ds(s * RPT, RPT)], acc_out.at[c, pl.ds(s * RPT, RPT)])


@functools.partial(
    pl.kernel,
    out_type=jax.ShapeDtypeStruct((EQ * OUT,), jnp.float32),
    mesh=_mesh,
    compiler_params=_sc_params,
    scratch_types=[
        pltpu.VMEM((N * 4,), jnp.float32),
        pltpu.VMEM((EPT,), jnp.int32),
        pltpu.VMEM((EPT,), jnp.int32),
        pltpu.VMEM((EPT * OUT,), jnp.float32),
    ],
)
def _k7_classify(ptab_f, qs_f, qd_f, out, ptab_v, qs_v, qd_v, out_v):
    w = _wid()
    pltpu.sync_copy(ptab_f, ptab_v)
    pltpu.sync_copy(qs_f.at[pl.ds(w * EPT, EPT)], qs_v)
    pltpu.sync_copy(qd_f.at[pl.ds(w * EPT, EPT)], qd_v)

    one = jnp.ones((VB,), jnp.float32)

    def body(i, _):
        s16 = qs_v[pl.ds(i * VB, VB)] * 4
        d16 = qd_v[pl.ds(i * VB, VB)] * 4
        a0 = plsc.load_gather(ptab_v, [s16])
        a1 = plsc.load_gather(ptab_v, [s16 + 1])
        b0 = plsc.load_gather(ptab_v, [d16 + 2])
        b1 = plsc.load_gather(ptab_v, [d16 + 3])
        out_v[pl.ds(i * VB, VB)] = one / (one + jnp.exp(-(a0 + b0)))
        out_v[pl.ds(EPT + i * VB, VB)] = one / (one + jnp.exp(-(a1 + b1)))
        return 0

    lax.fori_loop(0, NV, body, 0)
    pltpu.sync_copy(out_v.at[pl.ds(0, EPT)], out.at[pl.ds(w * EPT, EPT)])
    pltpu.sync_copy(out_v.at[pl.ds(EPT, EPT)], out.at[pl.ds(EQ + w * EPT, EPT)])


BN = 2000
GRID = N // BN


def _norm_from(parts):
    deg = jnp.sum(parts[0], axis=0)
    return lax.rsqrt(jnp.clip(deg, 1.0, None))


_HIST_SPEC = pl.BlockSpec((1, NW, BN), lambda j: (j, 0, 0))
_ACC_SPEC = pl.BlockSpec((NC, BN, H), lambda j: (0, j, 0))


def _k2_body(x_ref, hs_ref, w_ref, t_ref):
    ns = _norm_from(hs_ref[...])
    t_ref[...] = jnp.dot(
        x_ref[...] * ns[:, None], w_ref[...], preferred_element_type=jnp.float32
    )


def _k2_matmul(x, hs_part, W1):
    return pl.pallas_call(
        _k2_body,
        grid=(GRID,),
        in_specs=[
            pl.BlockSpec((BN, D), lambda j: (j, 0)),
            _HIST_SPEC,
            pl.BlockSpec((D, H), lambda j: (0, 0)),
        ],
        out_specs=pl.BlockSpec((BN, H), lambda j: (j, 0)),
        out_shape=jax.ShapeDtypeStruct((N, H), jnp.float32),
    )(x, hs_part, W1)


def _k4_body(acc_ref, hd_ref, hs_ref, b_ref, w_ref, t_ref):
    a = acc_ref[0] + acc_ref[1]
    nd = _norm_from(hd_ref[...])
    h = jnp.maximum(a * nd[:, None] + b_ref[...], 0.0)
    ns = _norm_from(hs_ref[...])
    t_ref[...] = jnp.dot(
        h * ns[:, None], w_ref[...], preferred_element_type=jnp.float32
    )


def _k4_mid(acc, hd_part, hs_part, b1, W2):
    return pl.pallas_call(
        _k4_body,
        grid=(GRID,),
        in_specs=[
            _ACC_SPEC,
            _HIST_SPEC,
            _HIST_SPEC,
            pl.BlockSpec((1, H), lambda j: (0, 0)),
            pl.BlockSpec((H, H), lambda j: (0, 0)),
        ],
        out_specs=pl.BlockSpec((BN, H), lambda j: (j, 0)),
        out_shape=jax.ShapeDtypeStruct((N, H), jnp.float32),
    )(acc, hd_part, hs_part, b1, W2)


def _k6_body(acc_ref, hd_ref, b_ref, wc1_ref, wc2_ref, bc_ref, h_ref, p_ref):
    a = acc_ref[0] + acc_ref[1]
    nd = _norm_from(hd_ref[...])
    h = jnp.maximum(a * nd[:, None] + b_ref[...], 0.0)
    h_ref[...] = h
    p1 = jnp.dot(h, wc1_ref[...], preferred_element_type=jnp.float32) + bc_ref[...]
    p2 = jnp.dot(h, wc2_ref[...], preferred_element_type=jnp.float32)
    p_ref[...] = jnp.concatenate([p1, p2], axis=1)


def _k6_final(acc, hd_part, b2, Wc1, Wc2, bc):
    return pl.pallas_call(
        _k6_body,
        grid=(GRID,),
        in_specs=[
            _ACC_SPEC,
            _HIST_SPEC,
            pl.BlockSpec((1, H), lambda j: (0, 0)),
            pl.BlockSpec((H, OUT), lambda j: (0, 0)),
            pl.BlockSpec((H, OUT), lambda j: (0, 0)),
            pl.BlockSpec((1, OUT), lambda j: (0, 0)),
        ],
        out_specs=[
            pl.BlockSpec((BN, H), lambda j: (j, 0)),
            pl.BlockSpec((BN, 2 * OUT), lambda j: (j, 0)),
        ],
        out_shape=[
            jax.ShapeDtypeStruct((N, H), jnp.float32),
            jax.ShapeDtypeStruct((N, 2 * OUT), jnp.float32),
        ],
    )(acc, hd_part, b2, Wc1, Wc2, bc)


def _pad_edges(idx, fill):
    chunks = idx.reshape(NW, EPT)
    pad = jnp.full((NW, EPTP - EPT), fill, jnp.int32)
    return jnp.concatenate([chunks, pad], axis=1).reshape(NW, NB, BB)


def kernel(x, graph_edge_index, edge_index, W1, b1, W2, b2, Wc, bc):
    src = graph_edge_index[0]
    dst = graph_edge_index[1]
    src3 = _pad_edges(src, 0)
    dst3 = _pad_edges(dst, TRASH)
    zrows = jnp.zeros((RPT, D), jnp.float32)

    hs_flat, hd_flat = _k1_hist(src, dst)
    hs_part = hs_flat.reshape(NW, GRID, BN).transpose(1, 0, 2)
    hd_part = hd_flat.reshape(NW, GRID, BN).transpose(1, 0, 2)

    t1 = _k2_matmul(x, hs_part, W1)
    acc1 = _k3_scatter(t1, src3, dst3, zrows)
    t2 = _k4_mid(acc1, hd_part, hs_part, b1.reshape(1, H), W2)
    acc2 = _k3_scatter(t2, src3, dst3, zrows)
    h2, ptab = _k6_final(
        acc2, hd_part, b2.reshape(1, H), Wc[:H], Wc[H:], bc.reshape(1, OUT)
    )
    pf = _k7_classify(ptab.reshape(N * 4), edge_index[0], edge_index[1])
    probs = jnp.stack([pf[:EQ], pf[EQ:]], axis=1)
    return (h2, probs)


# speedup vs baseline: 5.9057x; 1.0225x over previous
import functools

import jax
import jax.numpy as jnp
from jax import lax
from jax.experimental import pallas as pl
from jax.experimental.pallas import tpu as pltpu
from jax.experimental.pallas import tpu_sc as plsc

N = 10000
E = 320000
EQ = 320000
D = 128
H = 128
OUT = 2

NC = 2
NS = 16
NW = NC * NS
EPT = E // NW
BB = 128
NB = 80
CH = 40
EPTP = NB * BB
NA = 10112
RPT = NA // NS
VB = 16
NV = EPT // VB
TRASH = N

_mesh = plsc.VectorSubcoreMesh(
    core_axis_name="c", subcore_axis_name="s", num_cores=NC, num_subcores=NS
)
_sc_params = pltpu.CompilerParams(needs_layout_passes=False)


def _wid():
    return lax.axis_index("s") * NC + lax.axis_index("c")


@functools.partial(
    pl.kernel,
    out_type=(
        jax.ShapeDtypeStruct((NW * N,), jnp.float32),
        jax.ShapeDtypeStruct((NW * N,), jnp.float32),
    ),
    mesh=_mesh,
    compiler_params=_sc_params,
    scratch_types=[
        pltpu.VMEM((EPT,), jnp.int32),
        pltpu.VMEM((EPT,), jnp.int32),
        pltpu.VMEM((N,), jnp.float32),
        pltpu.VMEM((N,), jnp.float32),
    ],
)
def _k1_hist(src_f, dst_f, out_s, out_d, idx_s, idx_d, hist_s, hist_d):
    w = _wid()
    pltpu.sync_copy(src_f.at[pl.ds(w * EPT, EPT)], idx_s)
    pltpu.sync_copy(dst_f.at[pl.ds(w * EPT, EPT)], idx_d)

    z16 = jnp.zeros((VB,), jnp.float32)

    def zero_body(i, _):
        hist_s[pl.ds(i * VB, VB)] = z16
        hist_d[pl.ds(i * VB, VB)] = z16
        return 0

    lax.fori_loop(0, N // VB, zero_body, 0)

    ones16 = jnp.ones((VB,), jnp.float32)

    def body(i, _):
        s16 = idx_s[pl.ds(i * VB, VB)]
        d16 = idx_d[pl.ds(i * VB, VB)]
        plsc.addupdate_scatter(hist_s, [s16], ones16)
        plsc.addupdate_scatter(hist_d, [d16], ones16)
        return 0

    lax.fori_loop(0, NV, body, 0)

    pltpu.sync_copy(hist_s, out_s.at[pl.ds(w * N, N)])
    pltpu.sync_copy(hist_d, out_d.at[pl.ds(w * N, N)])


@functools.partial(
    pl.kernel,
    out_type=jax.ShapeDtypeStruct((NC, NA, D), jnp.float32),
    mesh=_mesh,
    compiler_params=_sc_params,
    scratch_types=[
        pltpu.VMEM((CH, BB), jnp.int32),
        pltpu.VMEM((CH, BB), jnp.int32),
        pltpu.VMEM((BB, D), jnp.float32),
        pltpu.VMEM((BB, D), jnp.float32),
        pltpu.VMEM_SHARED((NA, D), jnp.float32),
        pltpu.SemaphoreType.DMA,
    ],
)
def _k3_scatter(
    t, src3, dst3, zrows, acc_out,
    idx_s, idx_d, r0, r1, acc_sh, s0,
):
    c = lax.axis_index("c")
    s = lax.axis_index("s")
    w = s * NC + c

    pltpu.sync_copy(zrows, acc_sh.at[pl.ds(s * RPT, RPT)])
    plsc.subcore_barrier()

    def drain(buf):
        pltpu.make_async_copy(t.at[idx_s.at[0]], buf, s0).wait()

    def body(g, _):
        pltpu.sync_copy(src3.at[w, pl.ds(g * CH, CH)], idx_s)
        pltpu.sync_copy(dst3.at[w, pl.ds(g * CH, CH)], idx_d)
        for k in range(CH // 2):
            pltpu.sync_copy(t.at[idx_s.at[2 * k]], r0)
            if k > 0:
                drain(r1)
            pltpu.async_copy(r0, acc_sh.at[idx_d.at[2 * k]], s0, add=True)
            pltpu.sync_copy(t.at[idx_s.at[2 * k + 1]], r1)
            drain(r0)
            pltpu.async_copy(r1, acc_sh.at[idx_d.at[2 * k + 1]], s0, add=True)
        drain(r1)
        return 0

    lax.fori_loop(0, NB // CH, body, 0)
    plsc.subcore_barrier()

    pltpu.sync_copy(acc_sh.at[pl.ds(s * RPT, RPT)], acc_out.at[c, pl.ds(s * RPT, RPT)])


@functools.partial(
    pl.kernel,
    out_type=jax.ShapeDtypeStruct((EQ * OUT,), jnp.float32),
    mesh=_mesh,
    compiler_params=_sc_params,
    scratch_types=[
        pltpu.VMEM((N * 4,), jnp.float32),
        pltpu.VMEM((EPT,), jnp.int32),
        pltpu.VMEM((EPT,), jnp.int32),
        pltpu.VMEM((EPT * OUT,), jnp.float32),
    ],
)
def _k7_classify(ptab_f, qs_f, qd_f, out, ptab_v, qs_v, qd_v, out_v):
    w = _wid()
    pltpu.sync_copy(ptab_f, ptab_v)
    pltpu.sync_copy(qs_f.at[pl.ds(w * EPT, EPT)], qs_v)
    pltpu.sync_copy(qd_f.at[pl.ds(w * EPT, EPT)], qd_v)

    one = jnp.ones((VB,), jnp.float32)

    def body(i, _):
        s16 = qs_v[pl.ds(i * VB, VB)] * 4
        d16 = qd_v[pl.ds(i * VB, VB)] * 4
        a0 = plsc.load_gather(ptab_v, [s16])
        a1 = plsc.load_gather(ptab_v, [s16 + 1])
        b0 = plsc.load_gather(ptab_v, [d16 + 2])
        b1 = plsc.load_gather(ptab_v, [d16 + 3])
        out_v[pl.ds(i * VB, VB)] = one / (one + jnp.exp(-(a0 + b0)))
        out_v[pl.ds(EPT + i * VB, VB)] = one / (one + jnp.exp(-(a1 + b1)))
        return 0

    lax.fori_loop(0, NV, body, 0)
    pltpu.sync_copy(out_v.at[pl.ds(0, EPT)], out.at[pl.ds(w * EPT, EPT)])
    pltpu.sync_copy(out_v.at[pl.ds(EPT, EPT)], out.at[pl.ds(EQ + w * EPT, EPT)])


BN = 2000
GRID = N // BN


def _norm_from(parts):
    deg = jnp.sum(parts[0], axis=0)
    return lax.rsqrt(jnp.clip(deg, 1.0, None))


_HIST_SPEC = pl.BlockSpec((1, NW, BN), lambda j: (j, 0, 0))
_ACC_SPEC = pl.BlockSpec((NC, BN, H), lambda j: (0, j, 0))


def _k2_body(x_ref, hs_ref, w_ref, t_ref):
    ns = _norm_from(hs_ref[...])
    t_ref[...] = jnp.dot(
        x_ref[...] * ns[:, None], w_ref[...], preferred_element_type=jnp.float32
    )


def _k2_matmul(x, hs_part, W1):
    return pl.pallas_call(
        _k2_body,
        grid=(GRID,),
        in_specs=[
            pl.BlockSpec((BN, D), lambda j: (j, 0)),
            _HIST_SPEC,
            pl.BlockSpec((D, H), lambda j: (0, 0)),
        ],
        out_specs=pl.BlockSpec((BN, H), lambda j: (j, 0)),
        out_shape=jax.ShapeDtypeStruct((N, H), jnp.float32),
    )(x, hs_part, W1)


def _k4_body(acc_ref, hd_ref, hs_ref, b_ref, w_ref, t_ref):
    a = acc_ref[0] + acc_ref[1]
    nd = _norm_from(hd_ref[...])
    h = jnp.maximum(a * nd[:, None] + b_ref[...], 0.0)
    ns = _norm_from(hs_ref[...])
    t_ref[...] = jnp.dot(
        h * ns[:, None], w_ref[...], preferred_element_type=jnp.float32
    )


def _k4_mid(acc, hd_part, hs_part, b1, W2):
    return pl.pallas_call(
        _k4_body,
        grid=(GRID,),
        in_specs=[
            _ACC_SPEC,
            _HIST_SPEC,
            _HIST_SPEC,
            pl.BlockSpec((1, H), lambda j: (0, 0)),
            pl.BlockSpec((H, H), lambda j: (0, 0)),
        ],
        out_specs=pl.BlockSpec((BN, H), lambda j: (j, 0)),
        out_shape=jax.ShapeDtypeStruct((N, H), jnp.float32),
    )(acc, hd_part, hs_part, b1, W2)


def _k6_body(acc_ref, hd_ref, b_ref, wc1_ref, wc2_ref, bc_ref, h_ref, p_ref):
    a = acc_ref[0] + acc_ref[1]
    nd = _norm_from(hd_ref[...])
    h = jnp.maximum(a * nd[:, None] + b_ref[...], 0.0)
    h_ref[...] = h
    p1 = jnp.dot(h, wc1_ref[...], preferred_element_type=jnp.float32) + bc_ref[...]
    p2 = jnp.dot(h, wc2_ref[...], preferred_element_type=jnp.float32)
    p_ref[...] = jnp.concatenate([p1, p2], axis=1)


def _k6_final(acc, hd_part, b2, Wc1, Wc2, bc):
    return pl.pallas_call(
        _k6_body,
        grid=(GRID,),
        in_specs=[
            _ACC_SPEC,
            _HIST_SPEC,
            pl.BlockSpec((1, H), lambda j: (0, 0)),
            pl.BlockSpec((H, OUT), lambda j: (0, 0)),
            pl.BlockSpec((H, OUT), lambda j: (0, 0)),
            pl.BlockSpec((1, OUT), lambda j: (0, 0)),
        ],
        out_specs=[
            pl.BlockSpec((BN, H), lambda j: (j, 0)),
            pl.BlockSpec((BN, 2 * OUT), lambda j: (j, 0)),
        ],
        out_shape=[
            jax.ShapeDtypeStruct((N, H), jnp.float32),
            jax.ShapeDtypeStruct((N, 2 * OUT), jnp.float32),
        ],
    )(acc, hd_part, b2, Wc1, Wc2, bc)


def _pad_edges(idx, fill):
    chunks = idx.reshape(NW, EPT)
    pad = jnp.full((NW, EPTP - EPT), fill, jnp.int32)
    return jnp.concatenate([chunks, pad], axis=1).reshape(NW, NB, BB)


def kernel(x, graph_edge_index, edge_index, W1, b1, W2, b2, Wc, bc):
    src = graph_edge_index[0]
    dst = graph_edge_index[1]
    src3 = _pad_edges(src, 0)
    dst3 = _pad_edges(dst, TRASH)
    zrows = jnp.zeros((RPT, D), jnp.float32)

    hs_flat, hd_flat = _k1_hist(src, dst)
    hs_part = hs_flat.reshape(NW, GRID, BN).transpose(1, 0, 2)
    hd_part = hd_flat.reshape(NW, GRID, BN).transpose(1, 0, 2)

    t1 = _k2_matmul(x, hs_part, W1)
    acc1 = _k3_scatter(t1, src3, dst3, zrows)
    t2 = _k4_mid(acc1, hd_part, hs_part, b1.reshape(1, H), W2)
    acc2 = _k3_scatter(t2, src3, dst3, zrows)
    h2, ptab = _k6_final(
        acc2, hd_part, b2.reshape(1, H), Wc[:H], Wc[H:], bc.reshape(1, OUT)
    )
    pf = _k7_classify(ptab.reshape(N * 4), edge_index[0], edge_index[1])
    probs = jnp.stack([pf[:EQ], pf[EQ:]], axis=1)
    return (h2, probs)


# speedup vs baseline: 6.1264x; 1.0374x over previous
import functools

import jax
import jax.numpy as jnp
from jax import lax
from jax.experimental import pallas as pl
from jax.experimental.pallas import tpu as pltpu
from jax.experimental.pallas import tpu_sc as plsc

N = 10000
E = 320000
EQ = 320000
D = 128
H = 128
OUT = 2

NC = 2
NS = 16
NW = NC * NS
EPT = E // NW
BB = 128
NB = 80
CH = 40
EPTP = NB * BB
NA = 10112
RPT = NA // NS
VB = 16
NV = EPT // VB
TRASH = N

_mesh = plsc.VectorSubcoreMesh(
    core_axis_name="c", subcore_axis_name="s", num_cores=NC, num_subcores=NS
)
_sc_params = pltpu.CompilerParams(needs_layout_passes=False)


def _wid():
    return lax.axis_index("s") * NC + lax.axis_index("c")


@functools.partial(
    pl.kernel,
    out_type=(
        jax.ShapeDtypeStruct((NW * N,), jnp.float32),
        jax.ShapeDtypeStruct((NW * N,), jnp.float32),
    ),
    mesh=_mesh,
    compiler_params=_sc_params,
    scratch_types=[
        pltpu.VMEM((EPT,), jnp.int32),
        pltpu.VMEM((EPT,), jnp.int32),
        pltpu.VMEM((N,), jnp.float32),
        pltpu.VMEM((N,), jnp.float32),
    ],
)
def _k1_hist(src_f, dst_f, out_s, out_d, idx_s, idx_d, hist_s, hist_d):
    w = _wid()
    pltpu.sync_copy(src_f.at[pl.ds(w * EPT, EPT)], idx_s)
    pltpu.sync_copy(dst_f.at[pl.ds(w * EPT, EPT)], idx_d)

    z16 = jnp.zeros((VB,), jnp.float32)

    def zero_body(i, _):
        hist_s[pl.ds(i * VB, VB)] = z16
        hist_d[pl.ds(i * VB, VB)] = z16
        return 0

    lax.fori_loop(0, N // VB, zero_body, 0)

    ones16 = jnp.ones((VB,), jnp.float32)

    def body(i, _):
        s16 = idx_s[pl.ds(i * VB, VB)]
        d16 = idx_d[pl.ds(i * VB, VB)]
        plsc.addupdate_scatter(hist_s, [s16], ones16)
        plsc.addupdate_scatter(hist_d, [d16], ones16)
        return 0

    lax.fori_loop(0, NV, body, 0)

    pltpu.sync_copy(hist_s, out_s.at[pl.ds(w * N, N)])
    pltpu.sync_copy(hist_d, out_d.at[pl.ds(w * N, N)])


@functools.partial(
    pl.kernel,
    out_type=jax.ShapeDtypeStruct((NC, NA, D), jnp.float32),
    mesh=_mesh,
    compiler_params=_sc_params,
    scratch_types=[
        pltpu.VMEM((CH, BB), jnp.int32),
        pltpu.VMEM((CH, BB), jnp.int32),
        pltpu.VMEM((BB, D), jnp.float32),
        pltpu.VMEM((BB, D), jnp.float32),
        pltpu.VMEM_SHARED((NA, D), jnp.float32),
        pltpu.SemaphoreType.DMA,
        pltpu.SemaphoreType.DMA,
        pltpu.SemaphoreType.DMA,
    ],
)
def _k3_scatter(
    t, src3, dst3, zrows, acc_out,
    idx_s, idx_d, r0, r1, acc_sh, s0, gs0, gs1,
):
    c = lax.axis_index("c")
    s = lax.axis_index("s")
    w = s * NC + c

    pltpu.sync_copy(zrows, acc_sh.at[pl.ds(s * RPT, RPT)])
    plsc.subcore_barrier()

    def sdrain(buf):
        pltpu.make_async_copy(t.at[idx_s.at[0]], buf, s0).wait()

    def gwait(buf, sem):
        pltpu.make_async_copy(t.at[idx_s.at[0]], buf, sem).wait()

    def body(g, _):
        pltpu.sync_copy(src3.at[w, pl.ds(g * CH, CH)], idx_s)
        pltpu.sync_copy(dst3.at[w, pl.ds(g * CH, CH)], idx_d)
        pltpu.async_copy(t.at[idx_s.at[0]], r0, gs0)
        pltpu.async_copy(t.at[idx_s.at[1]], r1, gs1)
        for k in range(CH // 2 - 1):
            gwait(r0, gs0)
            pltpu.async_copy(r0, acc_sh.at[idx_d.at[2 * k]], s0, add=True)
            gwait(r1, gs1)
            sdrain(r0)
            pltpu.async_copy(t.at[idx_s.at[2 * k + 2]], r0, gs0)
            pltpu.async_copy(r1, acc_sh.at[idx_d.at[2 * k + 1]], s0, add=True)
            sdrain(r1)
            pltpu.async_copy(t.at[idx_s.at[2 * k + 3]], r1, gs1)
        gwait(r0, gs0)
        pltpu.async_copy(r0, acc_sh.at[idx_d.at[CH - 2]], s0, add=True)
        gwait(r1, gs1)
        sdrain(r0)
        pltpu.async_copy(r1, acc_sh.at[idx_d.at[CH - 1]], s0, add=True)
        sdrain(r1)
        return 0

    lax.fori_loop(0, NB // CH, body, 0)
    plsc.subcore_barrier()

    pltpu.sync_copy(acc_sh.at[pl.ds(s * RPT, RPT)], acc_out.at[c, pl.ds(s * RPT, RPT)])


@functools.partial(
    pl.kernel,
    out_type=jax.ShapeDtypeStruct((EQ * OUT,), jnp.float32),
    mesh=_mesh,
    compiler_params=_sc_params,
    scratch_types=[
        pltpu.VMEM((N * 4,), jnp.float32),
        pltpu.VMEM((EPT,), jnp.int32),
        pltpu.VMEM((EPT,), jnp.int32),
        pltpu.VMEM((EPT * OUT,), jnp.float32),
    ],
)
def _k7_classify(ptab_f, qs_f, qd_f, out, ptab_v, qs_v, qd_v, out_v):
    w = _wid()
    pltpu.sync_copy(ptab_f, ptab_v)
    pltpu.sync_copy(qs_f.at[pl.ds(w * EPT, EPT)], qs_v)
    pltpu.sync_copy(qd_f.at[pl.ds(w * EPT, EPT)], qd_v)

    one = jnp.ones((VB,), jnp.float32)

    def body(i, _):
        s16 = qs_v[pl.ds(i * VB, VB)] * 4
        d16 = qd_v[pl.ds(i * VB, VB)] * 4
        a0 = plsc.load_gather(ptab_v, [s16])
        a1 = plsc.load_gather(ptab_v, [s16 + 1])
        b0 = plsc.load_gather(ptab_v, [d16 + 2])
        b1 = plsc.load_gather(ptab_v, [d16 + 3])
        out_v[pl.ds(i * VB, VB)] = one / (one + jnp.exp(-(a0 + b0)))
        out_v[pl.ds(EPT + i * VB, VB)] = one / (one + jnp.exp(-(a1 + b1)))
        return 0

    lax.fori_loop(0, NV, body, 0)
    pltpu.sync_copy(out_v.at[pl.ds(0, EPT)], out.at[pl.ds(w * EPT, EPT)])
    pltpu.sync_copy(out_v.at[pl.ds(EPT, EPT)], out.at[pl.ds(EQ + w * EPT, EPT)])


BN = 2000
GRID = N // BN


def _norm_from(parts):
    deg = jnp.sum(parts[0], axis=0)
    return lax.rsqrt(jnp.clip(deg, 1.0, None))


_HIST_SPEC = pl.BlockSpec((1, NW, BN), lambda j: (j, 0, 0))
_ACC_SPEC = pl.BlockSpec((NC, BN, H), lambda j: (0, j, 0))


def _k2_body(x_ref, hs_ref, w_ref, t_ref):
    ns = _norm_from(hs_ref[...])
    t_ref[...] = jnp.dot(
        x_ref[...] * ns[:, None], w_ref[...], preferred_element_type=jnp.float32
    )


def _k2_matmul(x, hs_part, W1):
    return pl.pallas_call(
        _k2_body,
        grid=(GRID,),
        in_specs=[
            pl.BlockSpec((BN, D), lambda j: (j, 0)),
            _HIST_SPEC,
            pl.BlockSpec((D, H), lambda j: (0, 0)),
        ],
        out_specs=pl.BlockSpec((BN, H), lambda j: (j, 0)),
        out_shape=jax.ShapeDtypeStruct((N, H), jnp.float32),
    )(x, hs_part, W1)


def _k4_body(acc_ref, hd_ref, hs_ref, b_ref, w_ref, t_ref):
    a = acc_ref[0] + acc_ref[1]
    nd = _norm_from(hd_ref[...])
    h = jnp.maximum(a * nd[:, None] + b_ref[...], 0.0)
    ns = _norm_from(hs_ref[...])
    t_ref[...] = jnp.dot(
        h * ns[:, None], w_ref[...], preferred_element_type=jnp.float32
    )


def _k4_mid(acc, hd_part, hs_part, b1, W2):
    return pl.pallas_call(
        _k4_body,
        grid=(GRID,),
        in_specs=[
            _ACC_SPEC,
            _HIST_SPEC,
            _HIST_SPEC,
            pl.BlockSpec((1, H), lambda j: (0, 0)),
            pl.BlockSpec((H, H), lambda j: (0, 0)),
        ],
        out_specs=pl.BlockSpec((BN, H), lambda j: (j, 0)),
        out_shape=jax.ShapeDtypeStruct((N, H), jnp.float32),
    )(acc, hd_part, hs_part, b1, W2)


def _k6_body(acc_ref, hd_ref, b_ref, wc1_ref, wc2_ref, bc_ref, h_ref, p_ref):
    a = acc_ref[0] + acc_ref[1]
    nd = _norm_from(hd_ref[...])
    h = jnp.maximum(a * nd[:, None] + b_ref[...], 0.0)
    h_ref[...] = h
    p1 = jnp.dot(h, wc1_ref[...], preferred_element_type=jnp.float32) + bc_ref[...]
    p2 = jnp.dot(h, wc2_ref[...], preferred_element_type=jnp.float32)
    p_ref[...] = jnp.concatenate([p1, p2], axis=1)


def _k6_final(acc, hd_part, b2, Wc1, Wc2, bc):
    return pl.pallas_call(
        _k6_body,
        grid=(GRID,),
        in_specs=[
            _ACC_SPEC,
            _HIST_SPEC,
            pl.BlockSpec((1, H), lambda j: (0, 0)),
            pl.BlockSpec((H, OUT), lambda j: (0, 0)),
            pl.BlockSpec((H, OUT), lambda j: (0, 0)),
            pl.BlockSpec((1, OUT), lambda j: (0, 0)),
        ],
        out_specs=[
            pl.BlockSpec((BN, H), lambda j: (j, 0)),
            pl.BlockSpec((BN, 2 * OUT), lambda j: (j, 0)),
        ],
        out_shape=[
            jax.ShapeDtypeStruct((N, H), jnp.float32),
            jax.ShapeDtypeStruct((N, 2 * OUT), jnp.float32),
        ],
    )(acc, hd_part, b2, Wc1, Wc2, bc)


def _pad_edges(idx, fill):
    chunks = idx.reshape(NW, EPT)
    pad = jnp.full((NW, EPTP - EPT), fill, jnp.int32)
    return jnp.concatenate([chunks, pad], axis=1).reshape(NW, NB, BB)


def kernel(x, graph_edge_index, edge_index, W1, b1, W2, b2, Wc, bc):
    src = graph_edge_index[0]
    dst = graph_edge_index[1]
    src3 = _pad_edges(src, 0)
    dst3 = _pad_edges(dst, TRASH)
    zrows = jnp.zeros((RPT, D), jnp.float32)

    hs_flat, hd_flat = _k1_hist(src, dst)
    hs_part = hs_flat.reshape(NW, GRID, BN).transpose(1, 0, 2)
    hd_part = hd_flat.reshape(NW, GRID, BN).transpose(1, 0, 2)

    t1 = _k2_matmul(x, hs_part, W1)
    acc1 = _k3_scatter(t1, src3, dst3, zrows)
    t2 = _k4_mid(acc1, hd_part, hs_part, b1.reshape(1, H), W2)
    acc2 = _k3_scatter(t2, src3, dst3, zrows)
    h2, ptab = _k6_final(
        acc2, hd_part, b2.reshape(1, H), Wc[:H], Wc[H:], bc.reshape(1, OUT)
    )
    pf = _k7_classify(ptab.reshape(N * 4), edge_index[0], edge_index[1])
    probs = jnp.stack([pf[:EQ], pf[EQ:]], axis=1)
    return (h2, probs)
